# baseline (speedup 1.0000x reference)
import jax
import jax.numpy as jnp
from jax.experimental import pallas as pl
from jax.experimental.pallas import tpu as pltpu

N = 4096
F_IN = 128
F_OUT = 128
BK = 1024


def _support_kernel(x_ref, attn_ref, w_ref, alpha_ref, adj_ref,
                    sup_ref, sw_ref, aiw_ref):
    k = pl.program_id(0)

    @pl.when(k == 0)
    def _prologue():
        x = x_ref[...]
        logits = jnp.sum(x * attn_ref[...], axis=1, keepdims=True)
        m = jnp.max(logits)
        e = jnp.exp(logits - m)
        s = e / jnp.sum(e)
        w = w_ref[...].astype(jnp.bfloat16)
        sw_ref[...] = jnp.dot((x * s).astype(jnp.bfloat16), w,
                              preferred_element_type=jnp.float32
                              ).astype(jnp.bfloat16)
        aiw_ref[...] = alpha_ref[0, 0] * jnp.dot(
            x.astype(jnp.bfloat16), w, preferred_element_type=jnp.float32)

    rows = pl.ds(k * BK, BK)
    sup_ref[...] = jnp.dot(adj_ref[...].astype(jnp.bfloat16), sw_ref[...],
                           preferred_element_type=jnp.float32) + aiw_ref[rows, :]


def _output_kernel(sup_ref, inc_ref, out_ref):
    out_ref[...] = jnp.dot(inc_ref[...].astype(jnp.bfloat16),
                           sup_ref[...].astype(jnp.bfloat16),
                           preferred_element_type=jnp.float32)


@jax.jit
def _run(input, adj, incidence_matrix, weight, attn, alpha):
    attn_row = attn.reshape(1, F_IN)
    alpha2d = alpha.reshape(1, 1)
    grid = (N // BK,)
    support = pl.pallas_call(
        _support_kernel,
        grid=grid,
        in_specs=[
            pl.BlockSpec((N, F_IN), lambda k: (0, 0)),
            pl.BlockSpec((1, F_IN), lambda k: (0, 0)),
            pl.BlockSpec((F_IN, F_OUT), lambda k: (0, 0)),
            pl.BlockSpec((1, 1), lambda k: (0, 0)),
            pl.BlockSpec((BK, N), lambda k: (k, 0)),
        ],
        out_specs=pl.BlockSpec((BK, F_OUT), lambda k: (k, 0)),
        out_shape=jax.ShapeDtypeStruct((N, F_OUT), jnp.float32),
        scratch_shapes=[
            pltpu.VMEM((N, F_OUT), jnp.bfloat16),
            pltpu.VMEM((N, F_OUT), jnp.float32),
        ],
    )(input, attn_row, weight, alpha2d, adj)

    return pl.pallas_call(
        _output_kernel,
        grid=grid,
        in_specs=[
            pl.BlockSpec((N, F_OUT), lambda k: (0, 0)),
            pl.BlockSpec((BK, N), lambda k: (k, 0)),
        ],
        out_specs=pl.BlockSpec((BK, F_OUT), lambda k: (k, 0)),
        out_shape=jax.ShapeDtypeStruct((N, F_OUT), jnp.float32),
    )(support, incidence_matrix)


def kernel(input, adj, incidence_matrix, weight, attn, alpha):
    return _run(input, adj, incidence_matrix, weight, attn, alpha)


# speedup vs baseline: 1.1138x; 1.1138x over previous
import jax
import jax.numpy as jnp
from jax.experimental import pallas as pl
from jax.experimental.pallas import tpu as pltpu

N = 4096
F_IN = 128
F_OUT = 128
CB = 128
NC = N // CB
D = 8


def _fused_kernel(x_ref, attn_ref, w_ref, alpha_ref, adj_hbm, inc_hbm,
                  out_ref, adj_buf, inc_buf, sw_ref, aiw_ref, sup_ref,
                  adj_sem, inc_sem):
    def adj_copy(i, slot):
        return pltpu.make_async_copy(
            adj_hbm.at[pl.ds(i * CB, CB), :], adj_buf.at[slot],
            adj_sem.at[slot])

    def inc_copy(i, slot):
        return pltpu.make_async_copy(
            inc_hbm.at[pl.ds(i * CB, CB), :], inc_buf.at[slot],
            inc_sem.at[slot])

    for j in range(D):
        adj_copy(j, j).start()

    x = x_ref[...]
    logits = jnp.sum(x * attn_ref[...], axis=1, keepdims=True)
    m = jnp.max(logits)
    e = jnp.exp(logits - m)
    s = e / jnp.sum(e)
    w = w_ref[...].astype(jnp.bfloat16)
    sw_ref[...] = jnp.dot((x * s).astype(jnp.bfloat16), w,
                          preferred_element_type=jnp.float32
                          ).astype(jnp.bfloat16)
    aiw_ref[...] = alpha_ref[0, 0] * jnp.dot(
        x.astype(jnp.bfloat16), w, preferred_element_type=jnp.float32)

    def phase1(i, carry):
        slot = jax.lax.rem(i, D)
        adj_copy(i, slot).wait()
        rows = pl.ds(i * CB, CB)
        sup = jnp.dot(adj_buf[slot].astype(jnp.bfloat16), sw_ref[...],
                      preferred_element_type=jnp.float32) + aiw_ref[rows, :]
        sup_ref[rows, :] = sup.astype(jnp.bfloat16)

        nxt = i + D

        @pl.when(nxt < NC)
        def _():
            adj_copy(nxt, slot).start()

        @pl.when(nxt >= NC)
        def _():
            inc_copy(nxt - NC, slot).start()

        return carry

    jax.lax.fori_loop(0, NC, phase1, 0, unroll=2)

    def phase2(i, carry):
        slot = jax.lax.rem(i, D)
        inc_copy(i, slot).wait()
        rows = pl.ds(i * CB, CB)
        out_ref[rows, :] = jnp.dot(inc_buf[slot].astype(jnp.bfloat16),
                                   sup_ref[...],
                                   preferred_element_type=jnp.float32)
        nxt = i + D

        @pl.when(nxt < NC)
        def _():
            inc_copy(nxt, slot).start()

        return carry

    jax.lax.fori_loop(0, NC, phase2, 0, unroll=2)


@jax.jit
def _run(input, adj, incidence_matrix, weight, attn, alpha):
    attn_row = attn.reshape(1, F_IN)
    alpha2d = alpha.reshape(1, 1)
    vmem = pl.BlockSpec(memory_space=pltpu.MemorySpace.VMEM)
    hbm = pl.BlockSpec(memory_space=pltpu.MemorySpace.HBM)
    return pl.pallas_call(
        _fused_kernel,
        in_specs=[vmem, vmem, vmem, vmem, hbm, hbm],
        out_specs=vmem,
        out_shape=jax.ShapeDtypeStruct((N, F_OUT), jnp.float32),
        scratch_shapes=[
            pltpu.VMEM((D, CB, N), jnp.float32),
            pltpu.VMEM((D, CB, N), jnp.float32),
            pltpu.VMEM((N, F_OUT), jnp.bfloat16),
            pltpu.VMEM((N, F_OUT), jnp.float32),
            pltpu.VMEM((N, F_OUT), jnp.bfloat16),
            pltpu.SemaphoreType.DMA((D,)),
            pltpu.SemaphoreType.DMA((D,)),
        ],
    )(input, attn_row, weight, alpha2d, adj, incidence_matrix)


def kernel(input, adj, incidence_matrix, weight, attn, alpha):
    return _run(input, adj, incidence_matrix, weight, attn, alpha)
